# baseline (speedup 1.0000x reference)
import functools

import jax
import jax.numpy as jnp
from jax import lax
from jax.experimental import pallas as pl
from jax.experimental.pallas import tpu as pltpu
from jax.experimental.pallas import tpu_sc as plsc

_EDGES = [
    (0, 1), (20, 1), (2, 20), (4, 20), (8, 20), (12, 0), (16, 0), (3, 2),
    (5, 4), (9, 8), (13, 12), (17, 16), (6, 5), (10, 9), (14, 13), (18, 17),
    (7, 6), (11, 10), (15, 14), (19, 18), (21, 7), (22, 7), (23, 11), (24, 11),
]

_NJ = 25
_T = 300
_L = 16
_NW = 32
_NBUF = 3
_BSC = 256
_BS_TC = 8
_COLA = _T - 2 * _L + 4
_COLB = _T - _L
_COLS_MAIN = tuple(range(0, _COLA, _L))


def _sc_body(bone_hbm, center_hbm, out_hbm,
             buf, cbuf, bone_sem, cen_sem, out_sem):
    wid = lax.axis_index("s") * 2 + lax.axis_index("c")
    nch = bone_hbm.shape[1]
    per_w = _BSC // _NW
    base = wid * per_w

    def in_copies(g, slot):
        bidx = base + g
        return (
            pltpu.make_async_copy(bone_hbm.at[pl.ds(bidx, 1)],
                                  buf.at[pl.ds(slot, 1)], bone_sem.at[slot]),
            pltpu.make_async_copy(center_hbm.at[pl.ds(bidx, 1)],
                                  cbuf.at[pl.ds(slot, 1)], cen_sem.at[slot]),
        )

    def out_copy(g, slot):
        bidx = base + g
        return pltpu.make_async_copy(
            buf.at[pl.ds(slot, 1)], out_hbm.at[pl.ds(bidx, 1)],
            out_sem.at[slot])

    def step(g, _):
        slot = g - (g // _NBUF) * _NBUF
        bone_cp, cen_cp = in_copies(g, slot)
        bone_cp.wait()
        cen_cp.wait()

        for c in range(nch):
            for col in _COLS_MAIN:
                cv = cbuf[slot, c, pl.ds(col, _L)]
                vals = {1: cv}
                buf[slot, c, 1, pl.ds(col, _L)] = cv
                for v1, v2 in _EDGES:
                    v = buf[slot, c, v1, pl.ds(col, _L)] + vals[v2]
                    vals[v1] = v
                    buf[slot, c, v1, pl.ds(col, _L)] = v
            ca = cbuf[slot, c, pl.ds(_COLA, _L)]
            cb = cbuf[slot, c, pl.ds(_COLB, _L)]
            vals = {1: (ca, cb)}
            buf[slot, c, 1, pl.ds(_COLA, _L)] = ca
            buf[slot, c, 1, pl.ds(_COLB, _L)] = cb
            for v1, v2 in _EDGES:
                a = buf[slot, c, v1, pl.ds(_COLA, _L)]
                b = buf[slot, c, v1, pl.ds(_COLB, _L)]
                va = a + vals[v2][0]
                vb = b + vals[v2][1]
                vals[v1] = (va, vb)
                buf[slot, c, v1, pl.ds(_COLA, _L)] = va
                buf[slot, c, v1, pl.ds(_COLB, _L)] = vb

        out_copy(g, slot).start()

        @pl.when(g >= 1)
        def _():
            prev = g - 1
            out_copy(prev, prev - (prev // _NBUF) * _NBUF).wait()

        @pl.when(g + 2 < per_w)
        def _():
            nxt = g + 2
            nslot = nxt - (nxt // _NBUF) * _NBUF
            bone_np, cen_np = in_copies(nxt, nslot)
            bone_np.start()
            cen_np.start()

        return _

    for g0 in range(2):
        bone_cp, cen_cp = in_copies(g0, g0)
        bone_cp.start()
        cen_cp.start()

    lax.fori_loop(0, per_w, step, None)

    last = per_w - 1
    out_copy(last, last - (last // _NBUF) * _NBUF).wait()


def _stitch_body(sc_ref, tc_ref, out_ref):
    out_ref[...] = sc_ref[...]


def _tc_body(bone_ref, cen_ref, out_ref):
    cv = cen_ref[...]
    vals = {1: cv}
    out_ref[:, :, 1, :] = cv
    for v1, v2 in _EDGES:
        v = bone_ref[:, :, v1, :] + vals[v2]
        vals[v1] = v
        out_ref[:, :, v1, :] = v


def kernel(bone, center):
    b, ch, nj, t = bone.shape

    mesh = plsc.VectorSubcoreMesh(core_axis_name="c", subcore_axis_name="s")
    sc_k = functools.partial(
        pl.kernel,
        out_type=jax.ShapeDtypeStruct((_BSC, ch, nj, t), jnp.float32),
        mesh=mesh,
        compiler_params=pltpu.CompilerParams(use_tc_tiling_on_sc=True),
        scratch_types=[
            pltpu.VMEM((_NBUF, ch, nj, t), jnp.float32),
            pltpu.VMEM((_NBUF, ch, t), jnp.float32),
            pltpu.SemaphoreType.DMA((_NBUF,)),
            pltpu.SemaphoreType.DMA((_NBUF,)),
            pltpu.SemaphoreType.DMA((_NBUF,)),
        ],
    )(_sc_body)
    sc_out = sc_k(bone, center)

    ntc = b - _BSC
    off = _BSC // _BS_TC
    tc_out = pl.pallas_call(
        _tc_body,
        grid=(ntc // _BS_TC,),
        in_specs=[
            pl.BlockSpec((_BS_TC, ch, nj, t), lambda i: (off + i, 0, 0, 0)),
            pl.BlockSpec((_BS_TC, ch, t), lambda i: (off + i, 0, 0)),
        ],
        out_specs=pl.BlockSpec((_BS_TC, ch, nj, t), lambda i: (off + i, 0, 0, 0)),
        out_shape=jax.ShapeDtypeStruct((b, ch, nj, t), jnp.float32),
        compiler_params=pltpu.CompilerParams(
            dimension_semantics=("arbitrary",)),
    )(bone, center)

    return pl.pallas_call(
        _stitch_body,
        grid=(_BSC // _BS_TC,),
        in_specs=[
            pl.BlockSpec((_BS_TC, ch, nj, t), lambda i: (i, 0, 0, 0)),
            pl.BlockSpec((_BS_TC, ch, nj, t), lambda i: (i, 0, 0, 0)),
        ],
        out_specs=pl.BlockSpec((_BS_TC, ch, nj, t), lambda i: (i, 0, 0, 0)),
        out_shape=jax.ShapeDtypeStruct((b, ch, nj, t), jnp.float32),
        input_output_aliases={1: 0},
    )(sc_out, tc_out)


# speedup vs baseline: 1.8070x; 1.8070x over previous
import functools

import jax
import jax.numpy as jnp
from jax import lax
from jax.experimental import pallas as pl
from jax.experimental.pallas import tpu as pltpu
from jax.experimental.pallas import tpu_sc as plsc

_EDGES = [
    (0, 1), (20, 1), (2, 20), (4, 20), (8, 20), (12, 0), (16, 0), (3, 2),
    (5, 4), (9, 8), (13, 12), (17, 16), (6, 5), (10, 9), (14, 13), (18, 17),
    (7, 6), (11, 10), (15, 14), (19, 18), (21, 7), (22, 7), (23, 11), (24, 11),
]

_NJ = 25
_T = 300
_L = 16
_NW = 32
_NBUF = 3
_BSC = 128
_BT = 128
_COLA = _T - 2 * _L + 4
_COLB = _T - _L
_COLS_MAIN = tuple(range(0, _COLA, _L))


def _sc_body(bone_hbm, center_hbm, out_hbm,
             buf, cbuf, bone_sem, cen_sem, out_sem):
    wid = lax.axis_index("s") * 2 + lax.axis_index("c")
    nch = bone_hbm.shape[1]
    per_w = _BSC // _NW
    base = wid * per_w

    def in_copies(g, slot):
        bidx = base + g
        return (
            pltpu.make_async_copy(bone_hbm.at[pl.ds(bidx, 1)],
                                  buf.at[pl.ds(slot, 1)], bone_sem.at[slot]),
            pltpu.make_async_copy(center_hbm.at[pl.ds(bidx, 1)],
                                  cbuf.at[pl.ds(slot, 1)], cen_sem.at[slot]),
        )

    def out_copy(g, slot):
        bidx = base + g
        return pltpu.make_async_copy(
            buf.at[pl.ds(slot, 1)], out_hbm.at[pl.ds(bidx, 1)],
            out_sem.at[slot])

    def step(g, _):
        slot = g - (g // _NBUF) * _NBUF
        bone_cp, cen_cp = in_copies(g, slot)
        bone_cp.wait()
        cen_cp.wait()

        for c in range(nch):
            for col in _COLS_MAIN:
                cv = cbuf[slot, c, pl.ds(col, _L)]
                vals = {1: cv}
                buf[slot, c, 1, pl.ds(col, _L)] = cv
                for v1, v2 in _EDGES:
                    v = buf[slot, c, v1, pl.ds(col, _L)] + vals[v2]
                    vals[v1] = v
                    buf[slot, c, v1, pl.ds(col, _L)] = v
            ca = cbuf[slot, c, pl.ds(_COLA, _L)]
            cb = cbuf[slot, c, pl.ds(_COLB, _L)]
            vals = {1: (ca, cb)}
            buf[slot, c, 1, pl.ds(_COLA, _L)] = ca
            buf[slot, c, 1, pl.ds(_COLB, _L)] = cb
            for v1, v2 in _EDGES:
                a = buf[slot, c, v1, pl.ds(_COLA, _L)]
                b = buf[slot, c, v1, pl.ds(_COLB, _L)]
                va = a + vals[v2][0]
                vb = b + vals[v2][1]
                vals[v1] = (va, vb)
                buf[slot, c, v1, pl.ds(_COLA, _L)] = va
                buf[slot, c, v1, pl.ds(_COLB, _L)] = vb

        out_copy(g, slot).start()

        @pl.when(g >= 1)
        def _():
            prev = g - 1
            out_copy(prev, prev - (prev // _NBUF) * _NBUF).wait()

        @pl.when(g + 2 < per_w)
        def _():
            nxt = g + 2
            nslot = nxt - (nxt // _NBUF) * _NBUF
            bone_np, cen_np = in_copies(nxt, nslot)
            bone_np.start()
            cen_np.start()

        return _

    for g0 in range(2):
        bone_cp, cen_cp = in_copies(g0, g0)
        bone_cp.start()
        cen_cp.start()

    lax.fori_loop(0, per_w, step, None)

    last = per_w - 1
    out_copy(last, last - (last // _NBUF) * _NBUF).wait()


def _tc_body(bone_ref, cen_ref, out_ref):
    cv = cen_ref[0]
    vals = {1: cv}
    out_ref[0, 1] = cv
    for v1, v2 in _EDGES:
        v = bone_ref[0, v1] + vals[v2]
        vals[v1] = v
        out_ref[0, v1] = v


def kernel(bone, center):
    b, ch, nj, t = bone.shape

    mesh = plsc.VectorSubcoreMesh(core_axis_name="c", subcore_axis_name="s")
    sc_k = functools.partial(
        pl.kernel,
        out_type=jax.ShapeDtypeStruct((_BSC, ch, nj, t), jnp.float32),
        mesh=mesh,
        compiler_params=pltpu.CompilerParams(use_tc_tiling_on_sc=True),
        scratch_types=[
            pltpu.VMEM((_NBUF, ch, nj, t), jnp.float32),
            pltpu.VMEM((_NBUF, ch, t), jnp.float32),
            pltpu.SemaphoreType.DMA((_NBUF,)),
            pltpu.SemaphoreType.DMA((_NBUF,)),
            pltpu.SemaphoreType.DMA((_NBUF,)),
        ],
    )(_sc_body)
    sc_out = sc_k(bone, center)

    bone_t = jnp.transpose(bone, (1, 2, 3, 0))
    center_t = jnp.transpose(center, (1, 2, 0))
    sc_out_t = jnp.transpose(sc_out, (1, 2, 3, 0))

    off = _BSC // _BT
    tc_out_t = pl.pallas_call(
        _tc_body,
        grid=((b - _BSC) // _BT, ch),
        in_specs=[
            pl.BlockSpec((1, nj, t, _BT), lambda j, c: (c, 0, 0, off + j)),
            pl.BlockSpec((1, t, _BT), lambda j, c: (c, 0, off + j)),
        ],
        out_specs=pl.BlockSpec((1, nj, t, _BT), lambda j, c: (c, 0, 0, off + j)),
        out_shape=jax.ShapeDtypeStruct((ch, nj, t, b), jnp.float32),
        compiler_params=pltpu.CompilerParams(
            dimension_semantics=("arbitrary", "arbitrary")),
    )(bone_t, center_t)

    out_t = lax.dynamic_update_slice(tc_out_t, sc_out_t, (0, 0, 0, 0))
    return jnp.transpose(out_t, (3, 0, 1, 2))


# speedup vs baseline: 3.2710x; 1.8102x over previous
import functools

import jax
import jax.numpy as jnp
from jax import lax
from jax.experimental import pallas as pl
from jax.experimental.pallas import tpu as pltpu
from jax.experimental.pallas import tpu_sc as plsc

_EDGES = [
    (0, 1), (20, 1), (2, 20), (4, 20), (8, 20), (12, 0), (16, 0), (3, 2),
    (5, 4), (9, 8), (13, 12), (17, 16), (6, 5), (10, 9), (14, 13), (18, 17),
    (7, 6), (11, 10), (15, 14), (19, 18), (21, 7), (22, 7), (23, 11), (24, 11),
]

_NJ = 25
_T = 300
_L = 16
_NW = 32
_NBUF = 3
_BSC = 128
_BT = 128
_COLA = _T - 2 * _L + 4
_COLB = _T - _L
_COLS_MAIN = tuple(range(0, _COLA, _L))


def _sc_body(bone_hbm, center_hbm, out_hbm,
             buf, cbuf, bone_sem, cen_sem, out_sem):
    wid = lax.axis_index("s") * 2 + lax.axis_index("c")
    nch = bone_hbm.shape[1]
    per_w = _BSC // _NW
    base = wid * per_w

    def in_copies(g, slot):
        bidx = base + g
        return (
            pltpu.make_async_copy(bone_hbm.at[pl.ds(bidx, 1)],
                                  buf.at[pl.ds(slot, 1)], bone_sem.at[slot]),
            pltpu.make_async_copy(center_hbm.at[pl.ds(bidx, 1)],
                                  cbuf.at[pl.ds(slot, 1)], cen_sem.at[slot]),
        )

    def out_copy(g, slot):
        bidx = base + g
        return pltpu.make_async_copy(
            buf.at[pl.ds(slot, 1)], out_hbm.at[pl.ds(bidx, 1)],
            out_sem.at[slot])

    def step(g, _):
        slot = g - (g // _NBUF) * _NBUF
        bone_cp, cen_cp = in_copies(g, slot)
        bone_cp.wait()
        cen_cp.wait()

        for c in range(nch):
            for col in _COLS_MAIN:
                cv = cbuf[slot, c, pl.ds(col, _L)]
                vals = {1: cv}
                buf[slot, c, 1, pl.ds(col, _L)] = cv
                for v1, v2 in _EDGES:
                    v = buf[slot, c, v1, pl.ds(col, _L)] + vals[v2]
                    vals[v1] = v
                    buf[slot, c, v1, pl.ds(col, _L)] = v
            ca = cbuf[slot, c, pl.ds(_COLA, _L)]
            cb = cbuf[slot, c, pl.ds(_COLB, _L)]
            vals = {1: (ca, cb)}
            buf[slot, c, 1, pl.ds(_COLA, _L)] = ca
            buf[slot, c, 1, pl.ds(_COLB, _L)] = cb
            for v1, v2 in _EDGES:
                a = buf[slot, c, v1, pl.ds(_COLA, _L)]
                b = buf[slot, c, v1, pl.ds(_COLB, _L)]
                va = a + vals[v2][0]
                vb = b + vals[v2][1]
                vals[v1] = (va, vb)
                buf[slot, c, v1, pl.ds(_COLA, _L)] = va
                buf[slot, c, v1, pl.ds(_COLB, _L)] = vb

        out_copy(g, slot).start()

        @pl.when(g >= 1)
        def _():
            prev = g - 1
            out_copy(prev, prev - (prev // _NBUF) * _NBUF).wait()

        @pl.when(g + 2 < per_w)
        def _():
            nxt = g + 2
            nslot = nxt - (nxt // _NBUF) * _NBUF
            bone_np, cen_np = in_copies(nxt, nslot)
            bone_np.start()
            cen_np.start()

        return _

    for g0 in range(2):
        bone_cp, cen_cp = in_copies(g0, g0)
        bone_cp.start()
        cen_cp.start()

    lax.fori_loop(0, per_w, step, None)

    last = per_w - 1
    out_copy(last, last - (last // _NBUF) * _NBUF).wait()


def _tc_body(bone_ref, cen_ref, out_ref):
    cv = cen_ref[0]
    vals = {1: cv}
    out_ref[0, 1] = cv
    for v1, v2 in _EDGES:
        v = bone_ref[0, v1] + vals[v2]
        vals[v1] = v
        out_ref[0, v1] = v


def kernel(bone, center):
    b, ch, nj, t = bone.shape

    mesh = plsc.VectorSubcoreMesh(core_axis_name="c", subcore_axis_name="s")
    sc_k = functools.partial(
        pl.kernel,
        out_type=jax.ShapeDtypeStruct((_BSC, ch, nj, t), jnp.float32),
        mesh=mesh,
        compiler_params=pltpu.CompilerParams(use_tc_tiling_on_sc=True),
        scratch_types=[
            pltpu.VMEM((_NBUF, ch, nj, t), jnp.float32),
            pltpu.VMEM((_NBUF, ch, t), jnp.float32),
            pltpu.SemaphoreType.DMA((_NBUF,)),
            pltpu.SemaphoreType.DMA((_NBUF,)),
            pltpu.SemaphoreType.DMA((_NBUF,)),
        ],
    )(_sc_body)
    bone_sc = lax.slice(bone, (0, 0, 0, 0), (_BSC, ch, nj, t))
    center_sc = lax.slice(center, (0, 0, 0), (_BSC, ch, t))
    sc_out = sc_k(bone_sc, center_sc)

    bone_t = jnp.transpose(bone, (1, 2, 3, 0))
    center_t = jnp.transpose(center, (1, 2, 0))
    sc_out_t = jnp.transpose(sc_out, (1, 2, 3, 0))

    off = _BSC // _BT
    tc_out_t = pl.pallas_call(
        _tc_body,
        grid=((b - _BSC) // _BT, ch),
        in_specs=[
            pl.BlockSpec((1, nj, t, _BT), lambda j, c: (c, 0, 0, off + j)),
            pl.BlockSpec((1, t, _BT), lambda j, c: (c, 0, off + j)),
        ],
        out_specs=pl.BlockSpec((1, nj, t, _BT), lambda j, c: (c, 0, 0, off + j)),
        out_shape=jax.ShapeDtypeStruct((ch, nj, t, b), jnp.float32),
        compiler_params=pltpu.CompilerParams(
            dimension_semantics=("arbitrary", "arbitrary")),
    )(bone_t, center_t)

    out_t = lax.dynamic_update_slice(tc_out_t, sc_out_t, (0, 0, 0, 0))
    return jnp.transpose(out_t, (3, 0, 1, 2))
